# baseline (speedup 1.0000x reference)
import functools
import math

import jax
import jax.numpy as jnp
from jax import lax
from jax.experimental import pallas as pl
from jax.experimental.pallas import tpu as pltpu
from jax.experimental.pallas import tpu_sc as plsc

_B, _T, _E = 1024, 200, 64
_ROWS = _B * _T
_NW = 32
_RPW = _ROWS // _NW
_CH = 128
_NCH = _RPW // _CH
_LANES = 16


def _pe_table():
    position = jnp.arange(_T, dtype=jnp.float32)[:, None]
    div_term = jnp.exp(
        jnp.arange(0, _E, 2, dtype=jnp.float32) * -(math.log(10000.0) / _E)
    )
    pe = jnp.zeros((_T, _E), dtype=jnp.float32)
    pe = pe.at[:, 0::2].set(jnp.sin(position * div_term))
    pe = pe.at[:, 1::2].set(jnp.cos(position * div_term))
    return pe


@functools.partial(
    pl.kernel,
    out_type=jax.ShapeDtypeStruct((_ROWS, _E), jnp.float32),
    mesh=plsc.VectorSubcoreMesh(core_axis_name="c", subcore_axis_name="s"),
    compiler_params=pltpu.CompilerParams(use_tc_tiling_on_sc=False),
    scratch_types=[
        pltpu.VMEM((_T, _E), jnp.float32),
        pltpu.VMEM((_CH,), jnp.int32),
        pltpu.VMEM((_CH, _E), jnp.float32),
        pltpu.SemaphoreType.DMA,
    ],
)
def _emb_kernel(idx_hbm, table_hbm, pe_hbm, out_hbm, pe_v, idx_v, rows_v, sem):
    wid = lax.axis_index("s") * 2 + lax.axis_index("c")
    base = wid * _RPW
    pltpu.sync_copy(pe_hbm, pe_v)

    def chunk_body(c, carry):
        row0 = base + c * _CH
        pltpu.sync_copy(idx_hbm.at[pl.ds(row0, _CH)], idx_v)
        pltpu.async_copy(table_hbm.at[idx_v], rows_v, sem).wait()

        def add_row(i, t):
            for j in range(_E // _LANES):
                sl = pl.ds(j * _LANES, _LANES)
                rows_v[i, sl] = rows_v[i, sl] + pe_v[t, sl]
            t = t + 1
            return lax.select(t >= _T, 0, t)

        lax.fori_loop(0, _CH, add_row, lax.rem(row0, _T), unroll=4)
        pltpu.sync_copy(rows_v, out_hbm.at[pl.ds(row0, _CH)])
        return carry

    lax.fori_loop(0, _NCH, chunk_body, 0)


def kernel(inputs, token_emb):
    b, t = inputs.shape
    idx_flat = inputs.reshape(-1)
    out = _emb_kernel(idx_flat, token_emb, _pe_table())
    return out.reshape(b, t, token_emb.shape[1])


# speedup vs baseline: 1.0626x; 1.0626x over previous
import functools
import math

import jax
import jax.numpy as jnp
from jax import lax
from jax.experimental import pallas as pl
from jax.experimental.pallas import tpu as pltpu
from jax.experimental.pallas import tpu_sc as plsc

_B, _T, _E = 1024, 200, 64
_ROWS = _B * _T
_NW = 32
_SPW = _B // _NW
_HALF = _T // 2


def _pe_table():
    position = jnp.arange(_T, dtype=jnp.float32)[:, None]
    div_term = jnp.exp(
        jnp.arange(0, _E, 2, dtype=jnp.float32) * -(math.log(10000.0) / _E)
    )
    pe = jnp.zeros((_T, _E), dtype=jnp.float32)
    pe = pe.at[:, 0::2].set(jnp.sin(position * div_term))
    pe = pe.at[:, 1::2].set(jnp.cos(position * div_term))
    return pe


@functools.partial(
    pl.kernel,
    out_type=jax.ShapeDtypeStruct((_ROWS, _E), jnp.float32),
    mesh=plsc.VectorSubcoreMesh(core_axis_name="c", subcore_axis_name="s"),
    compiler_params=pltpu.CompilerParams(use_tc_tiling_on_sc=False),
    scratch_types=[
        pltpu.VMEM((2, 2, _HALF), jnp.int32),
        pltpu.VMEM((2, _T, _E), jnp.float32),
        pltpu.SemaphoreType.DMA,
        pltpu.SemaphoreType.DMA,
        pltpu.SemaphoreType.DMA,
        pltpu.SemaphoreType.DMA,
    ],
)
def _emb_kernel(idx_hbm, table_hbm, pe_hbm, out_hbm,
                idx_v, rows_v, gsem0, gsem1, wsem0, wsem1):
    wid = lax.axis_index("s") * 2 + lax.axis_index("c")
    seq0 = wid * _SPW
    gsems = (gsem0, gsem1)
    wsems = (wsem0, wsem1)

    def start_seq(k, buf):
        seq = seq0 + k
        pltpu.sync_copy(idx_hbm.at[pl.ds(2 * seq, 2)], idx_v.at[buf])
        pltpu.sync_copy(pe_hbm, rows_v.at[buf])
        for j in range(2):
            pltpu.async_copy(
                table_hbm.at[idx_v.at[buf, j]],
                rows_v.at[buf, pl.ds(j * _HALF, _HALF)],
                gsems[buf],
                add=True,
            )

    def finish_seq(k, buf):
        seq = seq0 + k
        for j in range(2):
            pltpu.make_async_copy(
                table_hbm.at[idx_v.at[buf, j]],
                rows_v.at[buf, pl.ds(j * _HALF, _HALF)],
                gsems[buf],
            ).wait()
        pltpu.async_copy(rows_v.at[buf], out_hbm.at[pl.ds(seq * _T, _T)],
                         wsems[buf])

    def wait_write(k, buf):
        seq = seq0 + k
        pltpu.make_async_copy(rows_v.at[buf],
                              out_hbm.at[pl.ds(seq * _T, _T)],
                              wsems[buf]).wait()

    start_seq(0, 0)

    def seq_body(k, carry):
        buf = lax.rem(k, 2)

        def step(cur, other):
            @pl.when(k >= 1)
            def _():
                wait_write(k - 1, other)
            start_seq(k + 1, other)
            finish_seq(k, cur)

        @pl.when(buf == 0)
        def _():
            step(0, 1)

        @pl.when(buf == 1)
        def _():
            step(1, 0)
        return carry

    lax.fori_loop(0, _SPW - 1, seq_body, 0)
    last = _SPW - 1
    buf_last = last % 2
    wait_write(last - 1, 1 - buf_last)
    finish_seq(last, buf_last)
    wait_write(last, buf_last)


def kernel(inputs, token_emb):
    b, t = inputs.shape
    e = token_emb.shape[1]
    table_lin = token_emb.ravel().reshape(token_emb.shape)
    idx2 = inputs.reshape(-1).reshape(_ROWS // _HALF, _HALF)
    out = _emb_kernel(idx2, table_lin, _pe_table())
    return out.reshape(b, t, e)


# speedup vs baseline: 1.1401x; 1.0730x over previous
import functools
import math

import jax
import jax.numpy as jnp
from jax import lax
from jax.experimental import pallas as pl
from jax.experimental.pallas import tpu as pltpu
from jax.experimental.pallas import tpu_sc as plsc

_B, _T, _E = 1024, 200, 64
_ROWS = _B * _T
_NW = 32
_SPW = _B // _NW
_HALF = _T // 2
_LANES = 16
_NBUF = 4


def _pe_table():
    position = jnp.arange(_T, dtype=jnp.float32)[:, None]
    div_term = jnp.exp(
        jnp.arange(0, _E, 2, dtype=jnp.float32) * -(math.log(10000.0) / _E)
    )
    pe = jnp.zeros((_T, _E), dtype=jnp.float32)
    pe = pe.at[:, 0::2].set(jnp.sin(position * div_term))
    pe = pe.at[:, 1::2].set(jnp.cos(position * div_term))
    return pe


@functools.partial(
    pl.kernel,
    out_type=jax.ShapeDtypeStruct((_ROWS, _E), jnp.float32),
    mesh=plsc.VectorSubcoreMesh(core_axis_name="c", subcore_axis_name="s"),
    compiler_params=pltpu.CompilerParams(use_tc_tiling_on_sc=False),
    scratch_types=[
        pltpu.VMEM((_T, _E), jnp.float32),
        pltpu.VMEM((2 * _SPW, _HALF), jnp.int32),
        pltpu.VMEM((_NBUF, _T, _E), jnp.float32),
        pltpu.SemaphoreType.DMA,
        pltpu.SemaphoreType.DMA,
        pltpu.SemaphoreType.DMA,
        pltpu.SemaphoreType.DMA,
        pltpu.SemaphoreType.DMA,
        pltpu.SemaphoreType.DMA,
        pltpu.SemaphoreType.DMA,
        pltpu.SemaphoreType.DMA,
    ],
)
def _emb_kernel(idx_hbm, table_hbm, pe_hbm, out_hbm,
                pe_v, idx_v, rows_v, g0, g1, g2, g3, w0, w1, w2, w3):
    wid = lax.axis_index("s") * 2 + lax.axis_index("c")
    seq0 = wid * _SPW
    pltpu.sync_copy(pe_hbm, pe_v)
    pltpu.sync_copy(idx_hbm.at[pl.ds(2 * seq0, 2 * _SPW)], idx_v)
    gsems = (g0, g1, g2, g3)
    wsems = (w0, w1, w2, w3)

    def start_seq(k, buf):
        for j in range(2):
            pltpu.async_copy(
                table_hbm.at[idx_v.at[2 * k + j]],
                rows_v.at[buf, pl.ds(j * _HALF, _HALF)],
                gsems[buf],
            )

    def finish_seq(k, buf):
        seq = seq0 + k
        pltpu.make_async_copy(
            table_hbm.at[pl.ds(0, _T)], rows_v.at[buf], gsems[buf]
        ).wait()

        def add_row(i, carry):
            for j in range(_E // _LANES):
                sl = pl.ds(j * _LANES, _LANES)
                rows_v[buf, i, sl] = rows_v[buf, i, sl] + pe_v[i, sl]
            return carry

        lax.fori_loop(0, _T, add_row, 0, unroll=2)
        pltpu.async_copy(rows_v.at[buf], out_hbm.at[pl.ds(seq * _T, _T)],
                         wsems[buf])

    def wait_write(k, buf):
        seq = seq0 + k
        pltpu.make_async_copy(rows_v.at[buf],
                              out_hbm.at[pl.ds(seq * _T, _T)],
                              wsems[buf]).wait()

    for p in range(_NBUF - 1):
        start_seq(p, p)

    def seq_body(k, carry):
        buf = lax.rem(k, _NBUF)

        def step(cur):
            nxt = (cur + _NBUF - 1) % _NBUF

            @pl.when(k >= 1)
            def _():
                wait_write(k - 1, nxt)

            @pl.when(k + _NBUF - 1 < _SPW)
            def _():
                start_seq(k + _NBUF - 1, nxt)
            finish_seq(k, cur)

        for b in range(_NBUF):
            @pl.when(buf == b)
            def _(b=b):
                step(b)
        return carry

    lax.fori_loop(0, _SPW, seq_body, 0)
    wait_write(_SPW - 1, (_SPW - 1) % _NBUF)


def kernel(inputs, token_emb):
    b, t = inputs.shape
    e = token_emb.shape[1]
    idx2 = inputs.reshape(-1).reshape(_ROWS // _HALF, _HALF)
    out = _emb_kernel(idx2, token_emb, _pe_table())
    return out.reshape(b, t, e)


# speedup vs baseline: 1.2338x; 1.0822x over previous
import functools
import math

import jax
import jax.numpy as jnp
from jax import lax
from jax.experimental import pallas as pl
from jax.experimental.pallas import tpu as pltpu
from jax.experimental.pallas import tpu_sc as plsc

_B, _T, _E = 1024, 200, 64
_ROWS = _B * _T
_NW = 32
_SPW = _B // _NW
_HALF = _T // 2
_LANES = 16
_NBUF = 4


def _pe_table():
    position = jnp.arange(_T, dtype=jnp.float32)[:, None]
    div_term = jnp.exp(
        jnp.arange(0, _E, 2, dtype=jnp.float32) * -(math.log(10000.0) / _E)
    )
    pe = jnp.zeros((_T, _E), dtype=jnp.float32)
    pe = pe.at[:, 0::2].set(jnp.sin(position * div_term))
    pe = pe.at[:, 1::2].set(jnp.cos(position * div_term))
    return pe


@functools.partial(
    pl.kernel,
    out_type=jax.ShapeDtypeStruct((_ROWS, _E), jnp.float32),
    mesh=plsc.VectorSubcoreMesh(core_axis_name="c", subcore_axis_name="s"),
    compiler_params=pltpu.CompilerParams(use_tc_tiling_on_sc=False),
    scratch_types=[
        pltpu.VMEM((_T, _E), jnp.float32),
        pltpu.VMEM((2 * _SPW, _HALF), jnp.int32),
        pltpu.VMEM((_NBUF, _T, _E), jnp.float32),
        pltpu.SemaphoreType.DMA,
        pltpu.SemaphoreType.DMA,
        pltpu.SemaphoreType.DMA,
        pltpu.SemaphoreType.DMA,
        pltpu.SemaphoreType.DMA,
        pltpu.SemaphoreType.DMA,
        pltpu.SemaphoreType.DMA,
        pltpu.SemaphoreType.DMA,
    ],
)
def _emb_kernel(idx_hbm, table_hbm, pe_hbm, out_hbm,
                pe_v, idx_v, rows_v, g0, g1, g2, g3, w0, w1, w2, w3):
    wid = lax.axis_index("s") * 2 + lax.axis_index("c")
    seq0 = wid * _SPW
    pltpu.sync_copy(pe_hbm, pe_v)
    pltpu.sync_copy(idx_hbm.at[pl.ds(2 * seq0, 2 * _SPW)], idx_v)
    gsems = (g0, g1, g2, g3)
    wsems = (w0, w1, w2, w3)

    def start_seq(k, buf):
        for j in range(2):
            pltpu.async_copy(
                table_hbm.at[idx_v.at[2 * k + j]],
                rows_v.at[buf, pl.ds(j * _HALF, _HALF)],
                gsems[buf],
            )

    def finish_seq(k, buf):
        seq = seq0 + k
        pltpu.make_async_copy(
            table_hbm.at[pl.ds(0, _T)], rows_v.at[buf], gsems[buf]
        ).wait()

        def add_row(i, carry):
            for j in range(_E // _LANES):
                sl = pl.ds(j * _LANES, _LANES)
                rows_v[buf, i, sl] = rows_v[buf, i, sl] + pe_v[i, sl]
            return carry

        lax.fori_loop(0, _T, add_row, 0, unroll=2)
        pltpu.async_copy(rows_v.at[buf], out_hbm.at[pl.ds(seq * _T, _T)],
                         wsems[buf])

    def wait_write(k, buf):
        seq = seq0 + k
        pltpu.make_async_copy(rows_v.at[buf],
                              out_hbm.at[pl.ds(seq * _T, _T)],
                              wsems[buf]).wait()

    for p in range(_NBUF - 1):
        start_seq(p, p)

    def seq_body(k, carry):
        buf = lax.rem(k, _NBUF)

        def step(cur):
            nxt = (cur + _NBUF - 1) % _NBUF

            @pl.when(k >= 1)
            def _():
                wait_write(k - 1, nxt)

            @pl.when(k + _NBUF - 1 < _SPW)
            def _():
                start_seq(k + _NBUF - 1, nxt)
            finish_seq(k, cur)

        for b in range(_NBUF):
            @pl.when(buf == b)
            def _(b=b):
                step(b)
        return carry

    lax.fori_loop(0, _SPW, seq_body, 0)
    wait_write(_SPW - 1, (_SPW - 1) % _NBUF)


def kernel(inputs, token_emb):
    b, t = inputs.shape
    e = token_emb.shape[1]
    padded = jnp.pad(token_emb, ((0, 0), (0, 128 - e)))
    tblv = padded.reshape(-1, e)
    idx2 = (inputs.reshape(-1) * 2).reshape(_ROWS // _HALF, _HALF)
    out = _emb_kernel(idx2, tblv, _pe_table())
    return out.reshape(b, t, e)


# speedup vs baseline: 1.6388x; 1.3282x over previous
import functools
import math

import jax
import jax.numpy as jnp
from jax import lax
from jax.experimental import pallas as pl
from jax.experimental.pallas import tpu as pltpu
from jax.experimental.pallas import tpu_sc as plsc

_B, _T, _E = 1024, 200, 64
_ROWS = _B * _T
_NW = 32
_SPW = _B // _NW
_HALF = _T // 2
_LANES = 16
_NBUF = 4


def _pe_table():
    position = jnp.arange(_T, dtype=jnp.float32)[:, None]
    div_term = jnp.exp(
        jnp.arange(0, _E, 2, dtype=jnp.float32) * -(math.log(10000.0) / _E)
    )
    pe = jnp.zeros((_T, _E), dtype=jnp.float32)
    pe = pe.at[:, 0::2].set(jnp.sin(position * div_term))
    pe = pe.at[:, 1::2].set(jnp.cos(position * div_term))
    return pe


@functools.partial(
    pl.kernel,
    out_type=jax.ShapeDtypeStruct((_ROWS, _E), jnp.float32),
    mesh=plsc.VectorSubcoreMesh(core_axis_name="c", subcore_axis_name="s"),
    compiler_params=pltpu.CompilerParams(use_tc_tiling_on_sc=False),
    scratch_types=[
        pltpu.VMEM((_T, _E), jnp.float32),
        pltpu.VMEM((2 * _SPW, _HALF), jnp.int32),
        pltpu.VMEM((_NBUF, _T, _E), jnp.float32),
        pltpu.SemaphoreType.DMA,
        pltpu.SemaphoreType.DMA,
        pltpu.SemaphoreType.DMA,
        pltpu.SemaphoreType.DMA,
        pltpu.SemaphoreType.DMA,
        pltpu.SemaphoreType.DMA,
        pltpu.SemaphoreType.DMA,
        pltpu.SemaphoreType.DMA,
    ],
)
def _emb_kernel(idx_hbm, table_hbm, pe_hbm, out_hbm,
                pe_v, idx_v, rows_v, g0, g1, g2, g3, w0, w1, w2, w3):
    wid = lax.axis_index("s") * 2 + lax.axis_index("c")
    seq0 = wid * _SPW
    pltpu.sync_copy(pe_hbm, pe_v)
    pltpu.sync_copy(idx_hbm.at[pl.ds(2 * seq0, 2 * _SPW)], idx_v)
    gsems = (g0, g1, g2, g3)
    wsems = (w0, w1, w2, w3)

    def start_seq(k, buf):
        for j in range(2):
            pltpu.async_copy(
                table_hbm.at[idx_v.at[2 * k + j]],
                rows_v.at[buf, pl.ds(j * _HALF, _HALF)],
                gsems[buf],
            )

    def finish_seq(k, buf):
        seq = seq0 + k
        pltpu.make_async_copy(
            table_hbm.at[pl.ds(0, _T)], rows_v.at[buf], gsems[buf]
        ).wait()

        def add_row(i, carry):
            for j in range(_E // _LANES):
                sl = pl.ds(j * _LANES, _LANES)
                rows_v[buf, i, sl] = rows_v[buf, i, sl] + pe_v[i, sl]
            return carry

        lax.fori_loop(0, _T, add_row, 0, unroll=2)
        pltpu.async_copy(rows_v.at[buf], out_hbm.at[pl.ds(seq * _T, _T)],
                         wsems[buf])

    def wait_write(k, buf):
        seq = seq0 + k
        pltpu.make_async_copy(rows_v.at[buf],
                              out_hbm.at[pl.ds(seq * _T, _T)],
                              wsems[buf]).wait()

    for p in range(_NBUF - 1):
        start_seq(p, p)

    def seq_body(k, carry):
        buf = lax.rem(k, _NBUF)

        def step(cur):
            nxt = (cur + _NBUF - 1) % _NBUF

            @pl.when(k >= 1)
            def _():
                wait_write(k - 1, nxt)

            @pl.when(k + _NBUF - 1 < _SPW)
            def _():
                start_seq(k + _NBUF - 1, nxt)
            finish_seq(k, cur)

        for b in range(_NBUF):
            @pl.when(buf == b)
            def _(b=b):
                step(b)
        return carry

    lax.fori_loop(0, _SPW, seq_body, 0)
    wait_write(_SPW - 1, (_SPW - 1) % _NBUF)


_V = 1000000
_TBLK = 4096


def _transpose_body(in_ref, out_ref):
    blk = in_ref[...]
    out_ref[:, :_E] = blk.T
    out_ref[:, _E:] = jnp.zeros((_TBLK, 128 - _E), jnp.float32)


def _transpose_table(tbl_t):
    grid = (_V + _TBLK - 1) // _TBLK
    return pl.pallas_call(
        _transpose_body,
        grid=(grid,),
        in_specs=[pl.BlockSpec((_E, _TBLK), lambda i: (0, i))],
        out_specs=pl.BlockSpec((_TBLK, 128), lambda i: (i, 0)),
        out_shape=jax.ShapeDtypeStruct((_V, 128), jnp.float32),
    )(tbl_t)


def kernel(inputs, token_emb):
    b, t = inputs.shape
    e = token_emb.shape[1]
    padded = _transpose_table(token_emb.T)
    tblv = padded.reshape(-1, e)
    idx2 = (inputs.reshape(-1) * 2).reshape(_ROWS // _HALF, _HALF)
    out = _emb_kernel(idx2, tblv, _pe_table())
    return out.reshape(b, t, e)


# speedup vs baseline: 1.8675x; 1.1396x over previous
import functools
import math

import jax
import jax.numpy as jnp
from jax import lax
from jax.experimental import pallas as pl
from jax.experimental.pallas import tpu as pltpu
from jax.experimental.pallas import tpu_sc as plsc

_B, _T, _E = 1024, 200, 64
_ROWS = _B * _T
_NW = 32
_SPW = _B // _NW
_HALF = _T // 2
_LANES = 16
_NBUF = 4


def _pe_table():
    position = jnp.arange(_T, dtype=jnp.float32)[:, None]
    div_term = jnp.exp(
        jnp.arange(0, _E, 2, dtype=jnp.float32) * -(math.log(10000.0) / _E)
    )
    pe = jnp.zeros((_T, _E), dtype=jnp.float32)
    pe = pe.at[:, 0::2].set(jnp.sin(position * div_term))
    pe = pe.at[:, 1::2].set(jnp.cos(position * div_term))
    return pe


@functools.partial(
    pl.kernel,
    out_type=jax.ShapeDtypeStruct((_ROWS, _E), jnp.float32),
    mesh=plsc.VectorSubcoreMesh(core_axis_name="c", subcore_axis_name="s"),
    compiler_params=pltpu.CompilerParams(use_tc_tiling_on_sc=False),
    scratch_types=[
        pltpu.VMEM((_T, _E), jnp.float32),
        pltpu.VMEM((2 * _SPW, _HALF), jnp.int32),
        pltpu.VMEM((_NBUF, _T, _E), jnp.float32),
        pltpu.SemaphoreType.DMA,
        pltpu.SemaphoreType.DMA,
        pltpu.SemaphoreType.DMA,
        pltpu.SemaphoreType.DMA,
        pltpu.SemaphoreType.DMA,
        pltpu.SemaphoreType.DMA,
        pltpu.SemaphoreType.DMA,
        pltpu.SemaphoreType.DMA,
    ],
)
def _emb_kernel(idx_hbm, table_hbm, pe_hbm, out_hbm,
                pe_v, idx_v, rows_v, g0, g1, g2, g3, w0, w1, w2, w3):
    wid = lax.axis_index("s") * 2 + lax.axis_index("c")
    seq0 = wid * _SPW
    pltpu.sync_copy(pe_hbm, pe_v)
    pltpu.sync_copy(idx_hbm.at[pl.ds(2 * seq0, 2 * _SPW)], idx_v)
    gsems = (g0, g1, g2, g3)
    wsems = (w0, w1, w2, w3)

    def start_seq(k, buf):
        for j in range(2):
            pltpu.async_copy(
                table_hbm.at[idx_v.at[2 * k + j]],
                rows_v.at[buf, pl.ds(j * _HALF, _HALF)],
                gsems[buf],
            )

    def finish_seq(k, buf):
        seq = seq0 + k
        pltpu.make_async_copy(
            table_hbm.at[pl.ds(0, _T)], rows_v.at[buf], gsems[buf]
        ).wait()

        def add_row(i, carry):
            for j in range(_E // _LANES):
                sl = pl.ds(j * _LANES, _LANES)
                rows_v[buf, i, sl] = rows_v[buf, i, sl] + pe_v[i, sl]
            return carry

        lax.fori_loop(0, _T, add_row, 0, unroll=2)
        pltpu.async_copy(rows_v.at[buf], out_hbm.at[pl.ds(seq * _T, _T)],
                         wsems[buf])

    def wait_write(k, buf):
        seq = seq0 + k
        pltpu.make_async_copy(rows_v.at[buf],
                              out_hbm.at[pl.ds(seq * _T, _T)],
                              wsems[buf]).wait()

    for p in range(_NBUF - 1):
        start_seq(p, p)

    def seq_body(k, carry):
        buf = lax.rem(k, _NBUF)

        def step(cur):
            nxt = (cur + _NBUF - 1) % _NBUF

            @pl.when(k >= 1)
            def _():
                wait_write(k - 1, nxt)

            @pl.when(k + _NBUF - 1 < _SPW)
            def _():
                start_seq(k + _NBUF - 1, nxt)
            finish_seq(k, cur)

        for b in range(_NBUF):
            @pl.when(buf == b)
            def _(b=b):
                step(b)
        return carry

    lax.fori_loop(0, _SPW, seq_body, 0)
    wait_write(_SPW - 1, (_SPW - 1) % _NBUF)


_V = 1000000
_TBLK = 8192


def _transpose_body(in_ref, out_ref):
    blk = in_ref[...]
    out_ref[:, :_E] = blk.T
    out_ref[:, _E:] = jnp.zeros((_TBLK, 128 - _E), jnp.float32)


def _transpose_table(tbl_t):
    grid = (_V + _TBLK - 1) // _TBLK
    return pl.pallas_call(
        _transpose_body,
        grid=(grid,),
        in_specs=[pl.BlockSpec((_E, _TBLK), lambda i: (0, i))],
        out_specs=pl.BlockSpec((_TBLK, 128), lambda i: (i, 0)),
        out_shape=jax.ShapeDtypeStruct((_V, 128), jnp.float32),
        compiler_params=pltpu.CompilerParams(
            dimension_semantics=("parallel",),
        ),
    )(tbl_t)


def kernel(inputs, token_emb):
    b, t = inputs.shape
    e = token_emb.shape[1]
    padded = _transpose_table(token_emb.T)
    tblv = padded.reshape(-1, e)
    idx2 = (inputs.reshape(-1) * 2).reshape(_ROWS // _HALF, _HALF)
    out = _emb_kernel(idx2, tblv, _pe_table())
    return out.reshape(b, t, e)


# speedup vs baseline: 1.9481x; 1.0432x over previous
import functools
import math

import jax
import jax.numpy as jnp
from jax import lax
from jax.experimental import pallas as pl
from jax.experimental.pallas import tpu as pltpu
from jax.experimental.pallas import tpu_sc as plsc

_B, _T, _E = 1024, 200, 64
_ROWS = _B * _T
_NW = 32
_SPW = _B // _NW
_HALF = _T // 2
_LANES = 16
_NBUF = 4


def _pe_table():
    position = jnp.arange(_T, dtype=jnp.float32)[:, None]
    div_term = jnp.exp(
        jnp.arange(0, _E, 2, dtype=jnp.float32) * -(math.log(10000.0) / _E)
    )
    pe = jnp.zeros((_T, _E), dtype=jnp.float32)
    pe = pe.at[:, 0::2].set(jnp.sin(position * div_term))
    pe = pe.at[:, 1::2].set(jnp.cos(position * div_term))
    return pe


@functools.partial(
    pl.kernel,
    out_type=jax.ShapeDtypeStruct((_ROWS, _E), jnp.float32),
    mesh=plsc.VectorSubcoreMesh(core_axis_name="c", subcore_axis_name="s"),
    compiler_params=pltpu.CompilerParams(use_tc_tiling_on_sc=False),
    scratch_types=[
        pltpu.VMEM((_T, _E), jnp.float32),
        pltpu.VMEM((2 * _SPW, _HALF), jnp.int32),
        pltpu.VMEM((_NBUF, _T, _E), jnp.float32),
        pltpu.SemaphoreType.DMA,
        pltpu.SemaphoreType.DMA,
        pltpu.SemaphoreType.DMA,
        pltpu.SemaphoreType.DMA,
        pltpu.SemaphoreType.DMA,
        pltpu.SemaphoreType.DMA,
        pltpu.SemaphoreType.DMA,
        pltpu.SemaphoreType.DMA,
    ],
)
def _emb_kernel(idx_hbm, table_hbm, pe_hbm, out_hbm,
                pe_v, idx_v, rows_v, g0, g1, g2, g3, w0, w1, w2, w3):
    wid = lax.axis_index("s") * 2 + lax.axis_index("c")
    seq0 = wid * _SPW
    pltpu.sync_copy(pe_hbm, pe_v)
    pltpu.sync_copy(idx_hbm.at[pl.ds(2 * seq0, 2 * _SPW)], idx_v)
    gsems = (g0, g1, g2, g3)
    wsems = (w0, w1, w2, w3)

    def start_seq(k, buf):
        for j in range(2):
            pltpu.async_copy(
                table_hbm.at[idx_v.at[2 * k + j]],
                rows_v.at[buf, pl.ds(j * _HALF, _HALF)],
                gsems[buf],
            )

    def finish_seq(k, buf):
        seq = seq0 + k
        pltpu.make_async_copy(
            table_hbm.at[pl.ds(0, _T)], rows_v.at[buf], gsems[buf]
        ).wait()

        def add_row(i, carry):
            for j in range(_E // _LANES):
                sl = pl.ds(j * _LANES, _LANES)
                rows_v[buf, i, sl] = rows_v[buf, i, sl] + pe_v[i, sl]
            return carry

        lax.fori_loop(0, _T, add_row, 0, unroll=2)
        pltpu.async_copy(rows_v.at[buf], out_hbm.at[pl.ds(seq * _T, _T)],
                         wsems[buf])

    def wait_write(k, buf):
        seq = seq0 + k
        pltpu.make_async_copy(rows_v.at[buf],
                              out_hbm.at[pl.ds(seq * _T, _T)],
                              wsems[buf]).wait()

    for p in range(_NBUF - 1):
        start_seq(p, p)

    def seq_body(k, carry):
        buf = lax.rem(k, _NBUF)

        def step(cur):
            nxt = (cur + _NBUF - 1) % _NBUF

            @pl.when(k >= 1)
            def _():
                wait_write(k - 1, nxt)

            @pl.when(k + _NBUF - 1 < _SPW)
            def _():
                start_seq(k + _NBUF - 1, nxt)
            finish_seq(k, cur)

        for b in range(_NBUF):
            @pl.when(buf == b)
            def _(b=b):
                step(b)
        return carry

    lax.fori_loop(0, _SPW, seq_body, 0)
    wait_write(_SPW - 1, (_SPW - 1) % _NBUF)


_V = 1000000
_TBLK = 16384


def _transpose_body(in_ref, out_ref):
    blk = in_ref[...]
    out_ref[:, :_E] = blk.T
    out_ref[:, _E:] = jnp.zeros((_TBLK, 128 - _E), jnp.float32)


def _transpose_table(tbl_t):
    grid = (_V + _TBLK - 1) // _TBLK
    return pl.pallas_call(
        _transpose_body,
        grid=(grid,),
        in_specs=[pl.BlockSpec((_E, _TBLK), lambda i: (0, i))],
        out_specs=pl.BlockSpec((_TBLK, 128), lambda i: (i, 0)),
        out_shape=jax.ShapeDtypeStruct((_V, 128), jnp.float32),
        compiler_params=pltpu.CompilerParams(
            dimension_semantics=("parallel",),
        ),
    )(tbl_t)


def kernel(inputs, token_emb):
    b, t = inputs.shape
    e = token_emb.shape[1]
    padded = _transpose_table(token_emb.T)
    tblv = padded.reshape(-1, e)
    idx2 = (inputs.reshape(-1) * 2).reshape(_ROWS // _HALF, _HALF)
    out = _emb_kernel(idx2, tblv, _pe_table())
    return out.reshape(b, t, e)


# speedup vs baseline: 1.9739x; 1.0132x over previous
import functools
import math

import jax
import jax.numpy as jnp
from jax import lax
from jax.experimental import pallas as pl
from jax.experimental.pallas import tpu as pltpu
from jax.experimental.pallas import tpu_sc as plsc

_B, _T, _E = 1024, 200, 64
_ROWS = _B * _T
_NW = 32
_SPW = _B // _NW
_HALF = _T // 2
_LANES = 16
_NBUF = 4


def _pe_table():
    position = jnp.arange(_T, dtype=jnp.float32)[:, None]
    div_term = jnp.exp(
        jnp.arange(0, _E, 2, dtype=jnp.float32) * -(math.log(10000.0) / _E)
    )
    pe = jnp.zeros((_T, _E), dtype=jnp.float32)
    pe = pe.at[:, 0::2].set(jnp.sin(position * div_term))
    pe = pe.at[:, 1::2].set(jnp.cos(position * div_term))
    return pe


@functools.partial(
    pl.kernel,
    out_type=jax.ShapeDtypeStruct((_ROWS, _E), jnp.float32),
    mesh=plsc.VectorSubcoreMesh(core_axis_name="c", subcore_axis_name="s"),
    compiler_params=pltpu.CompilerParams(use_tc_tiling_on_sc=False),
    scratch_types=[
        pltpu.VMEM((_T, _E), jnp.float32),
        pltpu.VMEM((2 * _SPW, _HALF), jnp.int32),
        pltpu.VMEM((_NBUF, _T, _E), jnp.float32),
        pltpu.SemaphoreType.DMA,
        pltpu.SemaphoreType.DMA,
        pltpu.SemaphoreType.DMA,
        pltpu.SemaphoreType.DMA,
        pltpu.SemaphoreType.DMA,
        pltpu.SemaphoreType.DMA,
        pltpu.SemaphoreType.DMA,
        pltpu.SemaphoreType.DMA,
    ],
)
def _emb_kernel(idx_hbm, table_hbm, pe_hbm, out_hbm,
                pe_v, idx_v, rows_v, g0, g1, g2, g3, w0, w1, w2, w3):
    wid = lax.axis_index("s") * 2 + lax.axis_index("c")
    seq0 = wid * _SPW
    pltpu.sync_copy(pe_hbm, pe_v)
    pltpu.sync_copy(idx_hbm.at[pl.ds(2 * seq0, 2 * _SPW)], idx_v)
    gsems = (g0, g1, g2, g3)
    wsems = (w0, w1, w2, w3)

    def start_seq(k, buf):
        for j in range(2):
            pltpu.async_copy(
                table_hbm.at[idx_v.at[2 * k + j]],
                rows_v.at[buf, pl.ds(j * _HALF, _HALF)],
                gsems[buf],
            )

    def finish_seq(k, buf):
        seq = seq0 + k
        pltpu.make_async_copy(
            table_hbm.at[pl.ds(0, _T)], rows_v.at[buf], gsems[buf]
        ).wait()

        def add_row(i, carry):
            for j in range(_E // _LANES):
                sl = pl.ds(j * _LANES, _LANES)
                rows_v[buf, i, sl] = rows_v[buf, i, sl] + pe_v[i, sl]
            return carry

        lax.fori_loop(0, _T, add_row, 0, unroll=2)
        pltpu.async_copy(rows_v.at[buf], out_hbm.at[pl.ds(seq * _T, _T)],
                         wsems[buf])

    def wait_write(k, buf):
        seq = seq0 + k
        pltpu.make_async_copy(rows_v.at[buf],
                              out_hbm.at[pl.ds(seq * _T, _T)],
                              wsems[buf]).wait()

    for p in range(_NBUF - 1):
        start_seq(p, p)

    def seq_body(k, carry):
        buf = lax.rem(k, _NBUF)

        def step(cur):
            nxt = (cur + _NBUF - 1) % _NBUF

            @pl.when(k >= 1)
            def _():
                wait_write(k - 1, nxt)

            @pl.when(k + _NBUF - 1 < _SPW)
            def _():
                start_seq(k + _NBUF - 1, nxt)
            finish_seq(k, cur)

        for b in range(_NBUF):
            @pl.when(buf == b)
            def _(b=b):
                step(b)
        return carry

    lax.fori_loop(0, _SPW, seq_body, 0)
    wait_write(_SPW - 1, (_SPW - 1) % _NBUF)


_V = 1000000
_TBLK = 32768


def _transpose_body(in_ref, out_ref):
    blk = in_ref[...]
    out_ref[:, :_E] = blk.T
    out_ref[:, _E:] = jnp.zeros((_TBLK, 128 - _E), jnp.float32)


def _transpose_table(tbl_t):
    grid = (_V + _TBLK - 1) // _TBLK
    return pl.pallas_call(
        _transpose_body,
        grid=(grid,),
        in_specs=[pl.BlockSpec((_E, _TBLK), lambda i: (0, i))],
        out_specs=pl.BlockSpec((_TBLK, 128), lambda i: (i, 0)),
        out_shape=jax.ShapeDtypeStruct((_V, 128), jnp.float32),
        compiler_params=pltpu.CompilerParams(
            dimension_semantics=("parallel",),
        ),
    )(tbl_t)


def kernel(inputs, token_emb):
    b, t = inputs.shape
    e = token_emb.shape[1]
    padded = _transpose_table(token_emb.T)
    tblv = padded.reshape(-1, e)
    idx2 = (inputs.reshape(-1) * 2).reshape(_ROWS // _HALF, _HALF)
    out = _emb_kernel(idx2, tblv, _pe_table())
    return out.reshape(b, t, e)
